# baseline (speedup 1.0000x reference)
import functools

import jax
import jax.numpy as jnp
from jax import lax
from jax.experimental import pallas as pl
from jax.experimental.pallas import tpu as pltpu
from jax.experimental.pallas import tpu_sc as plsc

C, W, B, D = 128, 1024, 4, 64
TD, TW = D // 8, W // 128
ROWS_PER_SLAB = TD * TW * 8
N_SLABS = C * B
N_ROWS = N_SLABS * ROWS_PER_SLAB
NC, NS = 2, 16
N_WORKERS = NC * NS
SLABS_PER_WORKER = N_SLABS // N_WORKERS
CHUNK = 64


CS_PER_WORKER = C // N_WORKERS


def _sc_body(x_hbm, ww_hbm, wcs_hbm, o_hbm, pos_v, wcs_v, xb, ob):
    wid = lax.axis_index("s") * NC + lax.axis_index("c")

    def c_body(q, carry):
        cc = wid * CS_PER_WORKER + q
        pltpu.sync_copy(ww_hbm, pos_v)
        pltpu.sync_copy(wcs_hbm.at[cc], wcs_v)

        def build_row(ri, carry2):
            d = (ri // (TW * 8)) * 8 + (ri % 8)
            spl = wcs_v[d // 8, pl.ds((d % 8) * 16, 16)]
            for k in range(8):
                sl = pl.ds(k * 16, 16)
                pos_v[ri, sl] = pos_v[ri, sl] + spl
            return carry2

        lax.fori_loop(0, ROWS_PER_SLAB, build_row, 0)

        def b_body(bb, carry2):
            slab = cc * B + bb

            def chunk_body(ch, carry3):
                base = slab * ROWS_PER_SLAB + ch * CHUNK
                pltpu.sync_copy(x_hbm.at[pl.ds(base, CHUNK)], xb)

                def row_body(r, carry4):
                    ri = ch * CHUNK + r
                    for k in range(8):
                        sl = pl.ds(k * 16, 16)
                        ob[r, sl] = xb[r, sl] + pos_v[ri, sl]
                    return carry4

                lax.fori_loop(0, CHUNK, row_body, 0)
                pltpu.sync_copy(ob, o_hbm.at[pl.ds(base, CHUNK)])
                return carry3

            return lax.fori_loop(0, ROWS_PER_SLAB // CHUNK, chunk_body, 0)

        return lax.fori_loop(0, B, b_body, 0)

    lax.fori_loop(0, CS_PER_WORKER, c_body, 0)


def _to_rows(a4):
    a6 = a4.reshape(C, B, TD, 8, TW, 128).transpose(0, 1, 2, 4, 3, 5)
    return a6.reshape(N_ROWS, 128)


def _from_rows(r):
    a6 = r.reshape(C, B, TD, TW, 8, 128).transpose(0, 1, 2, 4, 3, 5)
    a4 = a6.reshape(C, B, D, W)
    return a4.transpose(0, 3, 1, 2)


def kernel(input_embeddings, W_word, W_char):
    x_t = jnp.transpose(input_embeddings, (0, 2, 3, 1))
    x2 = _to_rows(x_t)
    ww_t = W_word.T
    ww2 = ww_t.reshape(TD, 8, TW, 128).transpose(0, 2, 1, 3).reshape(ROWS_PER_SLAB, 128)
    wcs2 = jnp.repeat(W_char, 16, axis=1).reshape(C, 8, 128)

    mesh = plsc.VectorSubcoreMesh(core_axis_name="c", subcore_axis_name="s")
    run = pl.kernel(
        _sc_body,
        mesh=mesh,
        out_type=jax.ShapeDtypeStruct((N_ROWS, 128), jnp.float32),
        scratch_types=[
            pltpu.VMEM((ROWS_PER_SLAB, 128), jnp.float32),
            pltpu.VMEM((8, 128), jnp.float32),
            pltpu.VMEM((CHUNK, 128), jnp.float32),
            pltpu.VMEM((CHUNK, 128), jnp.float32),
        ],
    )
    out2 = run(x2, ww2, wcs2)
    return _from_rows(out2)


# speedup vs baseline: 1.0118x; 1.0118x over previous
import jax
import jax.numpy as jnp
from jax import lax
from jax.experimental import pallas as pl
from jax.experimental.pallas import tpu as pltpu
from jax.experimental.pallas import tpu_sc as plsc

C, W, B, D = 128, 1024, 4, 64
TD, TW = D // 8, W // 128
ROWS_PER_SLAB = TD * TW * 8
SLAB = ROWS_PER_SLAB * 128
N_SLABS = C * B
N_WORDS = N_SLABS * SLAB
NC, NS = 2, 16
N_WORKERS = NC * NS
CS_PER_WORKER = C // N_WORKERS
CHUNK_ROWS = 128
CHUNK = CHUNK_ROWS * 128


def _sc_body(x_hbm, ww_hbm, wcs_hbm, o_hbm, pos_v, wcs_v, xb, ob):
    wid = lax.axis_index("s") * NC + lax.axis_index("c")

    def c_body(q, carry):
        cc = wid * CS_PER_WORKER + q
        pltpu.sync_copy(ww_hbm, pos_v)
        pltpu.sync_copy(wcs_hbm.at[pl.ds(cc * 1024, 1024)], wcs_v)

        def build_row(ri, carry2):
            d = (ri // (TW * 8)) * 8 + (ri % 8)
            spl = wcs_v[pl.ds((d // 8) * 128 + (d % 8) * 16, 16)]
            base = ri * 128
            for k in range(8):
                sl = pl.ds(base + k * 16, 16)
                pos_v[sl] = pos_v[sl] + spl
            return carry2

        lax.fori_loop(0, ROWS_PER_SLAB, build_row, 0, unroll=4)

        def b_body(bb, carry2):
            slab = (cc * B + bb) * SLAB

            def chunk_body(ch, carry3):
                base = slab + ch * CHUNK
                pbase = ch * CHUNK
                pltpu.sync_copy(x_hbm.at[pl.ds(base, CHUNK)], xb)
                for i in range(CHUNK // 16):
                    sl = pl.ds(i * 16, 16)
                    ob[sl] = xb[sl] + pos_v[pl.ds(pbase + i * 16, 16)]
                pltpu.sync_copy(ob, o_hbm.at[pl.ds(base, CHUNK)])
                return carry3

            return lax.fori_loop(0, SLAB // CHUNK, chunk_body, 0)

        return lax.fori_loop(0, B, b_body, 0)

    lax.fori_loop(0, CS_PER_WORKER, c_body, 0)


def _to_rows(a4):
    a6 = a4.reshape(C, B, TD, 8, TW, 128).transpose(0, 1, 2, 4, 3, 5)
    return a6.reshape(N_WORDS)


def _from_rows(r):
    a6 = r.reshape(C, B, TD, TW, 8, 128).transpose(0, 1, 2, 4, 3, 5)
    a4 = a6.reshape(C, B, D, W)
    return a4.transpose(0, 3, 1, 2)


def kernel(input_embeddings, W_word, W_char):
    x_t = jnp.transpose(input_embeddings, (0, 2, 3, 1))
    x2 = _to_rows(x_t)
    ww_t = W_word.T
    ww2 = (
        ww_t.reshape(TD, 8, TW, 128).transpose(0, 2, 1, 3).reshape(SLAB)
    )
    wcs2 = jnp.repeat(W_char, 16, axis=1).reshape(C * 1024)

    mesh = plsc.VectorSubcoreMesh(core_axis_name="c", subcore_axis_name="s")
    run = pl.kernel(
        _sc_body,
        mesh=mesh,
        out_type=jax.ShapeDtypeStruct((N_WORDS,), jnp.float32),
        scratch_types=[
            pltpu.VMEM((SLAB,), jnp.float32),
            pltpu.VMEM((1024,), jnp.float32),
            pltpu.VMEM((CHUNK,), jnp.float32),
            pltpu.VMEM((CHUNK,), jnp.float32),
        ],
    )
    out2 = run(x2, ww2, wcs2)
    return _from_rows(out2)


# speedup vs baseline: 1.4096x; 1.3931x over previous
import jax
import jax.numpy as jnp
from jax import lax
from jax.experimental import pallas as pl
from jax.experimental.pallas import tpu as pltpu
from jax.experimental.pallas import tpu_sc as plsc

C, W, B, D = 128, 1024, 4, 64
TD, TW = D // 8, W // 128
ROWS_PER_SLAB = TD * TW * 8
SLAB = ROWS_PER_SLAB * 128
N_SLABS = C * B
N_WORDS = N_SLABS * SLAB
NC, NS = 2, 16
N_WORKERS = NC * NS
CS_PER_WORKER = C // N_WORKERS
CHUNK_ROWS = 64
CHUNK = CHUNK_ROWS * 128
CHUNKS_PER_C = B * SLAB // CHUNK


def _sc_body(x_hbm, ww_hbm, wcs_hbm, o_hbm, pos_v, wcs_v,
             xb0, xb1, ob0, ob1, si0, si1, so0, so1):
    wid = lax.axis_index("s") * NC + lax.axis_index("c")
    xbufs, obufs = (xb0, xb1), (ob0, ob1)
    isems, osems = (si0, si1), (so0, so1)

    def c_body(q, carry):
        cc = wid * CS_PER_WORKER + q
        pltpu.sync_copy(ww_hbm, pos_v)
        pltpu.sync_copy(wcs_hbm.at[pl.ds(cc * 1024, 1024)], wcs_v)

        def build_row(ri, carry2):
            d = (ri // (TW * 8)) * 8 + (ri % 8)
            spl = wcs_v[pl.ds((d // 8) * 128 + (d % 8) * 16, 16)]
            base = ri * 128
            for k in range(8):
                sl = pl.ds(base + k * 16, 16)
                pos_v[sl] = pos_v[sl] + spl
            return carry2

        lax.fori_loop(0, ROWS_PER_SLAB, build_row, 0, unroll=4)

        region = cc * B * SLAB
        for p in (0, 1):
            pltpu.async_copy(
                x_hbm.at[pl.ds(region + p * CHUNK, CHUNK)], xbufs[p], isems[p]
            )

        def pair_body(j, carry2):
            for p in (0, 1):
                i = j * 2 + p
                base = region + i * CHUNK
                pbase = (i % (SLAB // CHUNK)) * CHUNK
                xb, ob = xbufs[p], obufs[p]
                pltpu.make_async_copy(
                    x_hbm.at[pl.ds(base, CHUNK)], xb, isems[p]
                ).wait()

                @pl.when(j > 0)
                def _():
                    pltpu.make_async_copy(
                        ob, o_hbm.at[pl.ds(base, CHUNK)], osems[p]
                    ).wait()

                def vec_body(v, carry3):
                    sl = pl.ds(v * 16, 16)
                    ob[sl] = xb[sl] + pos_v[pl.ds(pbase + v * 16, 16)]
                    return carry3

                lax.fori_loop(0, CHUNK // 16, vec_body, 0, unroll=8)
                pltpu.async_copy(ob, o_hbm.at[pl.ds(base, CHUNK)], osems[p])

                @pl.when(i + 2 < CHUNKS_PER_C)
                def _():
                    pltpu.async_copy(
                        x_hbm.at[pl.ds(base + 2 * CHUNK, CHUNK)], xb, isems[p]
                    )

            return carry2

        lax.fori_loop(0, CHUNKS_PER_C // 2, pair_body, 0)
        for p in (0, 1):
            pltpu.make_async_copy(
                obufs[p], o_hbm.at[pl.ds(region, CHUNK)], osems[p]
            ).wait()
        return carry

    lax.fori_loop(0, CS_PER_WORKER, c_body, 0)


def _to_rows(a4):
    a6 = a4.reshape(C, B, TD, 8, TW, 128).transpose(0, 1, 2, 4, 3, 5)
    return a6.reshape(N_WORDS)


def _from_rows(r):
    a6 = r.reshape(C, B, TD, TW, 8, 128).transpose(0, 1, 2, 4, 3, 5)
    a4 = a6.reshape(C, B, D, W)
    return a4.transpose(0, 3, 1, 2)


def kernel(input_embeddings, W_word, W_char):
    x_t = jnp.transpose(input_embeddings, (0, 2, 3, 1))
    x2 = _to_rows(x_t)
    ww_t = W_word.T
    ww2 = (
        ww_t.reshape(TD, 8, TW, 128).transpose(0, 2, 1, 3).reshape(SLAB)
    )
    wcs2 = jnp.repeat(W_char, 16, axis=1).reshape(C * 1024)

    mesh = plsc.VectorSubcoreMesh(core_axis_name="c", subcore_axis_name="s")
    run = pl.kernel(
        _sc_body,
        mesh=mesh,
        out_type=jax.ShapeDtypeStruct((N_WORDS,), jnp.float32),
        scratch_types=[
            pltpu.VMEM((SLAB,), jnp.float32),
            pltpu.VMEM((1024,), jnp.float32),
            pltpu.VMEM((CHUNK,), jnp.float32),
            pltpu.VMEM((CHUNK,), jnp.float32),
            pltpu.VMEM((CHUNK,), jnp.float32),
            pltpu.VMEM((CHUNK,), jnp.float32),
            pltpu.SemaphoreType.DMA,
            pltpu.SemaphoreType.DMA,
            pltpu.SemaphoreType.DMA,
            pltpu.SemaphoreType.DMA,
        ],
    )
    out2 = run(x2, ww2, wcs2)
    return _from_rows(out2)


# speedup vs baseline: 6.7628x; 4.7977x over previous
import jax
import jax.numpy as jnp
from jax.experimental import pallas as pl


def _body(x_ref, ww_ref, wc_ref, o_ref):
    o_ref[...] = (
        x_ref[...]
        + ww_ref[...][None, None, :, :]
        + wc_ref[...][:, None, :, None]
    )


def kernel(input_embeddings, W_word, W_char):
    C, W, B, D = input_embeddings.shape
    x_t = jnp.transpose(input_embeddings, (0, 2, 3, 1))
    ww_t = W_word.T

    BC = 8
    out_t = pl.pallas_call(
        _body,
        grid=(C // BC,),
        in_specs=[
            pl.BlockSpec((BC, B, D, W), lambda i: (i, 0, 0, 0)),
            pl.BlockSpec((D, W), lambda i: (0, 0)),
            pl.BlockSpec((BC, D), lambda i: (i, 0)),
        ],
        out_specs=pl.BlockSpec((BC, B, D, W), lambda i: (i, 0, 0, 0)),
        out_shape=jax.ShapeDtypeStruct((C, B, D, W), jnp.float32),
    )(x_t, ww_t, W_char)
    return jnp.transpose(out_t, (0, 3, 1, 2))
